# initial kernel scaffold (unmeasured)
import jax
import jax.numpy as jnp
from jax import lax
from jax.experimental import pallas as pl
from jax.experimental.pallas import tpu as pltpu

T = 1024
D = 1024
V_LOCAL = 8192
HALF = T // 2


def kernel(ids, E):
    my_y = lax.axis_index("y")
    base = my_y * V_LOCAL
    local = ids - base
    in_range = (local >= 0) & (local < V_LOCAL)
    safe = jnp.where(in_range, local, 0)
    partial = jnp.where(in_range[:, None], E[safe], 0.0).astype(jnp.bfloat16)

    def body(p_ref, out_ref, ybuf, xbuf, sbuf, send_sems, recv_sems):
        my_x = lax.axis_index("x")
        my_yi = lax.axis_index("y")
        y_peer = (my_x, 1 - my_yi)
        x_peer = (1 - my_x, my_yi)

        barrier = pltpu.get_barrier_semaphore()
        for peer in (y_peer, x_peer):
            pl.semaphore_signal(
                barrier, inc=1, device_id=peer,
                device_id_type=pl.DeviceIdType.MESH,
            )
        pl.semaphore_wait(barrier, 2)

        row0 = my_x * HALF

        y_rdma = pltpu.make_async_remote_copy(
            src_ref=p_ref.at[pl.ds(row0, HALF), :],
            dst_ref=ybuf,
            send_sem=send_sems.at[0],
            recv_sem=recv_sems.at[0],
            device_id=y_peer,
            device_id_type=pl.DeviceIdType.MESH,
        )
        y_rdma.start()
        y_rdma.wait()

        sbuf[...] = p_ref[pl.ds(row0, HALF), :] + ybuf[...]

        x_rdma = pltpu.make_async_remote_copy(
            src_ref=sbuf,
            dst_ref=xbuf,
            send_sem=send_sems.at[1],
            recv_sem=recv_sems.at[1],
            device_id=x_peer,
            device_id_type=pl.DeviceIdType.MESH,
        )
        x_rdma.start()
        out_ref[pl.ds(row0, HALF), :] = sbuf[...].astype(jnp.float32)
        x_rdma.wait()
        other0 = (1 - my_x) * HALF
        out_ref[pl.ds(other0, HALF), :] = xbuf[...].astype(jnp.float32)

    out = pl.pallas_call(
        body,
        out_shape=jax.ShapeDtypeStruct((T, D), jnp.float32),
        in_specs=[pl.BlockSpec(memory_space=pltpu.VMEM)],
        out_specs=pl.BlockSpec(memory_space=pltpu.VMEM),
        scratch_shapes=[
            pltpu.VMEM((HALF, D), jnp.bfloat16),
            pltpu.VMEM((HALF, D), jnp.bfloat16),
            pltpu.VMEM((HALF, D), jnp.bfloat16),
            pltpu.SemaphoreType.DMA((2,)),
            pltpu.SemaphoreType.DMA((2,)),
        ],
        compiler_params=pltpu.CompilerParams(collective_id=0),
    )(partial)
    return out


# baseline (device time: 44803 ns/iter reference)
import jax
import jax.numpy as jnp
from jax import lax
from jax.experimental import pallas as pl
from jax.experimental.pallas import tpu as pltpu

T = 1024
D = 1024
V_LOCAL = 8192
HALF = T // 2


def kernel(ids, E):
    ids_v = ids.reshape(T, 1)

    def body(ids_s, ids_vref, e_ref, out_ref,
             gbuf, ysend, ybuf, xsend, xbuf,
             gsem, send_sems, recv_sems):
        my_x = lax.axis_index("x")
        my_y = lax.axis_index("y")
        y_peer = (my_x, 1 - my_y)
        x_peer = (1 - my_x, my_y)
        base = my_y * V_LOCAL
        row0 = my_x * HALF

        barrier = pltpu.get_barrier_semaphore()
        for peer in (y_peer, x_peer):
            pl.semaphore_signal(
                barrier, inc=1, device_id=peer,
                device_id_type=pl.DeviceIdType.MESH,
            )
        pl.semaphore_wait(barrier, 2)

        def issue(j, carry):
            idx = ids_s[row0 + j] - base
            ok = jnp.logical_and(idx >= 0, idx < V_LOCAL)
            safe = jnp.where(ok, idx, 0)
            pltpu.make_async_copy(
                e_ref.at[pl.ds(safe, 1), :],
                gbuf.at[pl.ds(j, 1), :],
                gsem,
            ).start()
            return carry

        lax.fori_loop(0, HALF, issue, 0, unroll=8)

        def drain(j, carry):
            pltpu.make_async_copy(
                e_ref.at[pl.ds(0, 1), :],
                gbuf.at[pl.ds(0, 1), :],
                gsem,
            ).wait()
            return carry

        lax.fori_loop(0, HALF, drain, 0, unroll=8)

        idxv = ids_vref[pl.ds(row0, HALF), :] - base
        maskv = jnp.logical_and(idxv >= 0, idxv < V_LOCAL)
        mine = jnp.where(maskv, gbuf[...], 0.0)
        ysend[...] = mine.astype(jnp.bfloat16)

        y_rdma = pltpu.make_async_remote_copy(
            src_ref=ysend,
            dst_ref=ybuf,
            send_sem=send_sems.at[0],
            recv_sem=recv_sems.at[0],
            device_id=y_peer,
            device_id_type=pl.DeviceIdType.MESH,
        )
        y_rdma.start()
        y_rdma.wait()

        summ = mine + ybuf[...].astype(jnp.float32)
        xsend[...] = summ.astype(jnp.bfloat16)

        x_rdma = pltpu.make_async_remote_copy(
            src_ref=xsend,
            dst_ref=xbuf,
            send_sem=send_sems.at[1],
            recv_sem=recv_sems.at[1],
            device_id=x_peer,
            device_id_type=pl.DeviceIdType.MESH,
        )
        x_rdma.start()
        out_ref[pl.ds(row0, HALF), :] = summ
        x_rdma.wait()
        other0 = (1 - my_x) * HALF
        out_ref[pl.ds(other0, HALF), :] = xbuf[...].astype(jnp.float32)

    out = pl.pallas_call(
        body,
        out_shape=jax.ShapeDtypeStruct((T, D), jnp.float32),
        in_specs=[
            pl.BlockSpec(memory_space=pltpu.SMEM),
            pl.BlockSpec(memory_space=pltpu.VMEM),
            pl.BlockSpec(memory_space=pltpu.HBM),
        ],
        out_specs=pl.BlockSpec(memory_space=pltpu.VMEM),
        scratch_shapes=[
            pltpu.VMEM((HALF, D), jnp.float32),
            pltpu.VMEM((HALF, D), jnp.bfloat16),
            pltpu.VMEM((HALF, D), jnp.bfloat16),
            pltpu.VMEM((HALF, D), jnp.bfloat16),
            pltpu.VMEM((HALF, D), jnp.bfloat16),
            pltpu.SemaphoreType.DMA,
            pltpu.SemaphoreType.DMA((2,)),
            pltpu.SemaphoreType.DMA((2,)),
        ],
        compiler_params=pltpu.CompilerParams(collective_id=0),
    )(ids, ids_v, E)
    return out


# device time: 34304 ns/iter; 1.3061x vs baseline; 1.3061x over previous
import jax
import jax.numpy as jnp
from jax import lax
from jax.experimental import pallas as pl
from jax.experimental.pallas import tpu as pltpu

T = 1024
D = 1024
V_LOCAL = 8192
HALF = T // 2
NC = 4
CH = HALF // NC


def kernel(ids, E):
    ids_v = ids.reshape(T, 1)

    def body(ids_s, ids_vref, e_ref, out_ref,
             gbuf, ysend, ybuf, xsend, xbuf,
             gsems, ys_sems, yr_sems, xs_sems, xr_sems):
        my_x = lax.axis_index("x")
        my_y = lax.axis_index("y")
        y_peer = (my_x, 1 - my_y)
        x_peer = (1 - my_x, my_y)
        base = my_y * V_LOCAL
        row0 = my_x * HALF
        other0 = (1 - my_x) * HALF

        barrier = pltpu.get_barrier_semaphore()
        for peer in (y_peer, x_peer):
            pl.semaphore_signal(
                barrier, inc=1, device_id=peer,
                device_id_type=pl.DeviceIdType.MESH,
            )
        pl.semaphore_wait(barrier, 2)

        def issue(j, carry):
            idx = ids_s[row0 + j] - base
            ok = jnp.logical_and(idx >= 0, idx < V_LOCAL)
            safe = jnp.where(ok, idx, 0)
            pltpu.make_async_copy(
                e_ref.at[pl.ds(safe, 1), :],
                gbuf.at[pl.ds(j, 1), :],
                gsems.at[j // CH],
            ).start()
            return carry

        lax.fori_loop(0, HALF, issue, 0, unroll=8)

        y_rdmas = []
        x_rdmas = []
        for c in range(NC):
            sl = pl.ds(c * CH, CH)

            def drain(j, carry, c=c):
                pltpu.make_async_copy(
                    e_ref.at[pl.ds(0, 1), :],
                    gbuf.at[pl.ds(0, 1), :],
                    gsems.at[c],
                ).wait()
                return carry

            lax.fori_loop(0, CH, drain, 0, unroll=8)

            idxv = ids_vref[pl.ds(row0 + c * CH, CH), :] - base
            maskv = jnp.logical_and(idxv >= 0, idxv < V_LOCAL)
            mine = jnp.where(maskv, gbuf[sl, :], 0.0)
            ysend[sl, :] = mine.astype(jnp.bfloat16)

            r = pltpu.make_async_remote_copy(
                src_ref=ysend.at[sl, :],
                dst_ref=ybuf.at[sl, :],
                send_sem=ys_sems.at[c],
                recv_sem=yr_sems.at[c],
                device_id=y_peer,
                device_id_type=pl.DeviceIdType.MESH,
            )
            r.start()
            y_rdmas.append(r)

        for c in range(NC):
            sl = pl.ds(c * CH, CH)
            y_rdmas[c].wait_recv()
            summ = ysend[sl, :] + ybuf[sl, :]
            xsend[sl, :] = summ
            out_ref[pl.ds(row0 + c * CH, CH), :] = summ.astype(jnp.float32)
            r = pltpu.make_async_remote_copy(
                src_ref=xsend.at[sl, :],
                dst_ref=xbuf.at[sl, :],
                send_sem=xs_sems.at[c],
                recv_sem=xr_sems.at[c],
                device_id=x_peer,
                device_id_type=pl.DeviceIdType.MESH,
            )
            r.start()
            x_rdmas.append(r)

        for c in range(NC):
            sl = pl.ds(c * CH, CH)
            x_rdmas[c].wait_recv()
            out_ref[pl.ds(other0 + c * CH, CH), :] = (
                xbuf[sl, :].astype(jnp.float32)
            )

        for c in range(NC):
            y_rdmas[c].wait_send()
            x_rdmas[c].wait_send()

    out = pl.pallas_call(
        body,
        out_shape=jax.ShapeDtypeStruct((T, D), jnp.float32),
        in_specs=[
            pl.BlockSpec(memory_space=pltpu.SMEM),
            pl.BlockSpec(memory_space=pltpu.VMEM),
            pl.BlockSpec(memory_space=pltpu.HBM),
        ],
        out_specs=pl.BlockSpec(memory_space=pltpu.VMEM),
        scratch_shapes=[
            pltpu.VMEM((HALF, D), jnp.float32),
            pltpu.VMEM((HALF, D), jnp.bfloat16),
            pltpu.VMEM((HALF, D), jnp.bfloat16),
            pltpu.VMEM((HALF, D), jnp.bfloat16),
            pltpu.VMEM((HALF, D), jnp.bfloat16),
            pltpu.SemaphoreType.DMA((NC,)),
            pltpu.SemaphoreType.DMA((NC,)),
            pltpu.SemaphoreType.DMA((NC,)),
            pltpu.SemaphoreType.DMA((NC,)),
            pltpu.SemaphoreType.DMA((NC,)),
        ],
        compiler_params=pltpu.CompilerParams(collective_id=0),
    )(ids, ids_v, E)
    return out


# device time: 32438 ns/iter; 1.3812x vs baseline; 1.0575x over previous
import jax
import jax.numpy as jnp
from jax import lax
from jax.experimental import pallas as pl
from jax.experimental.pallas import tpu as pltpu

T = 1024
D = 1024
V_LOCAL = 8192
HALF = T // 2
NC = 8
CH = HALF // NC


def kernel(ids, E):
    ids_v = ids.reshape(T, 1)

    def body(ids_s, ids_vref, e_ref, out_ref,
             gbuf, ysend, ybuf, xsend, xbuf,
             gsems, ys_sems, yr_sems, xs_sems, xr_sems):
        my_x = lax.axis_index("x")
        my_y = lax.axis_index("y")
        y_peer = (my_x, 1 - my_y)
        x_peer = (1 - my_x, my_y)
        base = my_y * V_LOCAL
        row0 = my_x * HALF
        other0 = (1 - my_x) * HALF

        barrier = pltpu.get_barrier_semaphore()
        for peer in (y_peer, x_peer):
            pl.semaphore_signal(
                barrier, inc=1, device_id=peer,
                device_id_type=pl.DeviceIdType.MESH,
            )
        pl.semaphore_wait(barrier, 2)

        cnts = []
        for c in range(NC):
            def issue(j, n, c=c):
                idx = ids_s[row0 + c * CH + j] - base
                ok = jnp.logical_and(idx >= 0, idx < V_LOCAL)
                safe = jnp.where(ok, idx, 0)

                @pl.when(ok)
                def _():
                    pltpu.make_async_copy(
                        e_ref.at[pl.ds(safe, 1), :],
                        gbuf.at[pl.ds(c * CH + j, 1), :],
                        gsems.at[c],
                    ).start()

                return n + ok.astype(jnp.int32)

            cnts.append(lax.fori_loop(0, CH, issue, 0, unroll=8))

        y_rdmas = []
        x_rdmas = []
        for c in range(NC):
            sl = pl.ds(c * CH, CH)

            def drain(j, carry, c=c):
                pltpu.make_async_copy(
                    e_ref.at[pl.ds(0, 1), :],
                    gbuf.at[pl.ds(0, 1), :],
                    gsems.at[c],
                ).wait()
                return carry

            lax.fori_loop(0, cnts[c], drain, 0)

            idxv = ids_vref[pl.ds(row0 + c * CH, CH), :] - base
            maskv = jnp.logical_and(idxv >= 0, idxv < V_LOCAL)
            mine = jnp.where(maskv, gbuf[sl, :], 0.0)
            ysend[sl, :] = mine.astype(jnp.bfloat16)

            r = pltpu.make_async_remote_copy(
                src_ref=ysend.at[sl, :],
                dst_ref=ybuf.at[sl, :],
                send_sem=ys_sems.at[c],
                recv_sem=yr_sems.at[c],
                device_id=y_peer,
                device_id_type=pl.DeviceIdType.MESH,
            )
            r.start()
            y_rdmas.append(r)

        for c in range(NC):
            sl = pl.ds(c * CH, CH)
            y_rdmas[c].wait_recv()
            summ = ysend[sl, :] + ybuf[sl, :]
            xsend[sl, :] = summ
            out_ref[pl.ds(row0 + c * CH, CH), :] = summ.astype(jnp.float32)
            r = pltpu.make_async_remote_copy(
                src_ref=xsend.at[sl, :],
                dst_ref=xbuf.at[sl, :],
                send_sem=xs_sems.at[c],
                recv_sem=xr_sems.at[c],
                device_id=x_peer,
                device_id_type=pl.DeviceIdType.MESH,
            )
            r.start()
            x_rdmas.append(r)

        for c in range(NC):
            sl = pl.ds(c * CH, CH)
            x_rdmas[c].wait_recv()
            out_ref[pl.ds(other0 + c * CH, CH), :] = (
                xbuf[sl, :].astype(jnp.float32)
            )

        for c in range(NC):
            y_rdmas[c].wait_send()
            x_rdmas[c].wait_send()

    out = pl.pallas_call(
        body,
        out_shape=jax.ShapeDtypeStruct((T, D), jnp.float32),
        in_specs=[
            pl.BlockSpec(memory_space=pltpu.SMEM),
            pl.BlockSpec(memory_space=pltpu.VMEM),
            pl.BlockSpec(memory_space=pltpu.HBM),
        ],
        out_specs=pl.BlockSpec(memory_space=pltpu.VMEM),
        scratch_shapes=[
            pltpu.VMEM((HALF, D), jnp.float32),
            pltpu.VMEM((HALF, D), jnp.bfloat16),
            pltpu.VMEM((HALF, D), jnp.bfloat16),
            pltpu.VMEM((HALF, D), jnp.bfloat16),
            pltpu.VMEM((HALF, D), jnp.bfloat16),
            pltpu.SemaphoreType.DMA((NC,)),
            pltpu.SemaphoreType.DMA((NC,)),
            pltpu.SemaphoreType.DMA((NC,)),
            pltpu.SemaphoreType.DMA((NC,)),
            pltpu.SemaphoreType.DMA((NC,)),
        ],
        compiler_params=pltpu.CompilerParams(collective_id=0),
    )(ids, ids_v, E)
    return out


# device time: 31221 ns/iter; 1.4350x vs baseline; 1.0390x over previous
import jax
import jax.numpy as jnp
from jax import lax
from jax.experimental import pallas as pl
from jax.experimental.pallas import tpu as pltpu

T = 1024
D = 1024
V_LOCAL = 8192
HALF = T // 2
NC = 8
CH = HALF // NC


def kernel(ids, E):
    base_out = lax.axis_index("y") * V_LOCAL
    safe_ids = jnp.clip(ids - base_out, 0, V_LOCAL - 1).astype(jnp.int32)
    ids_v = ids.reshape(T, 1)

    def body(safe_s, ids_vref, e_ref, out_ref,
             gbuf, ysend, ybuf, xsend, xbuf,
             gsems, ys_sems, yr_sems, xs_sems, xr_sems):
        my_x = lax.axis_index("x")
        my_y = lax.axis_index("y")
        y_peer = (my_x, 1 - my_y)
        x_peer = (1 - my_x, my_y)
        base = my_y * V_LOCAL
        row0 = my_x * HALF
        other0 = (1 - my_x) * HALF

        barrier = pltpu.get_barrier_semaphore()
        for peer in (y_peer, x_peer):
            pl.semaphore_signal(
                barrier, inc=1, device_id=peer,
                device_id_type=pl.DeviceIdType.MESH,
            )
        pl.semaphore_wait(barrier, 2)

        def issue_chunk(c):
            start = row0 + c * CH

            def issue(j, carry):
                pltpu.make_async_copy(
                    e_ref.at[pl.ds(safe_s[start + j], 1), :],
                    gbuf.at[pl.ds(c * CH + j, 1), :],
                    gsems.at[c],
                ).start()
                return carry

            lax.fori_loop(0, CH, issue, 0, unroll=16)

        y_rdmas = {}
        x_rdmas = {}

        def send_y(c):
            def drain(j, carry):
                pltpu.make_async_copy(
                    e_ref.at[pl.ds(0, 1), :],
                    gbuf.at[pl.ds(0, 1), :],
                    gsems.at[c],
                ).wait()
                return carry

            lax.fori_loop(0, CH, drain, 0, unroll=16)
            sl = pl.ds(c * CH, CH)
            idxv = ids_vref[pl.ds(row0 + c * CH, CH), :] - base
            maskv = jnp.logical_and(idxv >= 0, idxv < V_LOCAL)
            ysend[sl, :] = jnp.where(maskv, gbuf[sl, :], 0.0).astype(
                jnp.bfloat16
            )
            r = pltpu.make_async_remote_copy(
                src_ref=ysend.at[sl, :],
                dst_ref=ybuf.at[sl, :],
                send_sem=ys_sems.at[c],
                recv_sem=yr_sems.at[c],
                device_id=y_peer,
                device_id_type=pl.DeviceIdType.MESH,
            )
            r.start()
            y_rdmas[c] = r

        def process_y(c):
            sl = pl.ds(c * CH, CH)
            y_rdmas[c].wait_recv()
            summ = ysend[sl, :] + ybuf[sl, :]
            xsend[sl, :] = summ
            out_ref[pl.ds(row0 + c * CH, CH), :] = summ.astype(jnp.float32)
            r = pltpu.make_async_remote_copy(
                src_ref=xsend.at[sl, :],
                dst_ref=xbuf.at[sl, :],
                send_sem=xs_sems.at[c],
                recv_sem=xr_sems.at[c],
                device_id=x_peer,
                device_id_type=pl.DeviceIdType.MESH,
            )
            r.start()
            x_rdmas[c] = r

        def process_x(c):
            sl = pl.ds(c * CH, CH)
            x_rdmas[c].wait_recv()
            out_ref[pl.ds(other0 + c * CH, CH), :] = (
                xbuf[sl, :].astype(jnp.float32)
            )

        issue_chunk(0)
        for c in range(NC):
            if c + 1 < NC:
                issue_chunk(c + 1)
            send_y(c)
            if c >= 1:
                process_y(c - 1)
            if c >= 2:
                process_x(c - 2)
        process_y(NC - 1)
        process_x(NC - 2)
        process_x(NC - 1)

        for c in range(NC):
            y_rdmas[c].wait_send()
            x_rdmas[c].wait_send()

    out = pl.pallas_call(
        body,
        out_shape=jax.ShapeDtypeStruct((T, D), jnp.float32),
        in_specs=[
            pl.BlockSpec(memory_space=pltpu.SMEM),
            pl.BlockSpec(memory_space=pltpu.VMEM),
            pl.BlockSpec(memory_space=pltpu.HBM),
        ],
        out_specs=pl.BlockSpec(memory_space=pltpu.VMEM),
        scratch_shapes=[
            pltpu.VMEM((HALF, D), jnp.float32),
            pltpu.VMEM((HALF, D), jnp.bfloat16),
            pltpu.VMEM((HALF, D), jnp.bfloat16),
            pltpu.VMEM((HALF, D), jnp.bfloat16),
            pltpu.VMEM((HALF, D), jnp.bfloat16),
            pltpu.SemaphoreType.DMA((NC,)),
            pltpu.SemaphoreType.DMA((NC,)),
            pltpu.SemaphoreType.DMA((NC,)),
            pltpu.SemaphoreType.DMA((NC,)),
            pltpu.SemaphoreType.DMA((NC,)),
        ],
        compiler_params=pltpu.CompilerParams(collective_id=0),
    )(safe_ids, ids_v, E)
    return out
